# baseline (speedup 1.0000x reference)
import functools

import jax
import jax.numpy as jnp
from jax import lax
from jax.experimental import pallas as pl
from jax.experimental.pallas import tpu as pltpu
from jax.experimental.pallas import tpu_sc as plsc

_D = 256
_NE = 8192
_NT = 8192
_TM = 1024
_TN = 8192
_NI = _NT // _TM
_NJ = _NE // _TN


def _prep_body(f_ref, e_ref, fbf_ref, fsq_ref, ebf_ref, c_ref):
    f = f_ref[...]
    e = e_ref[...]
    norm = jnp.sqrt(jnp.sum(e * e, axis=1, keepdims=True))
    ehat = e / norm
    ebf_ref[...] = (ehat + ehat).astype(jnp.bfloat16)
    c_ref[...] = jnp.sum(ehat * ehat, axis=1, keepdims=True)
    fbf_ref[...] = f.astype(jnp.bfloat16)
    ones = jnp.ones((1, _D), dtype=jnp.float32)
    fsq_ref[...] = lax.dot_general(ones, f * f, (((1,), (1,)), ((), ())),
                                   preferred_element_type=jnp.float32,
                                   precision=lax.Precision.HIGHEST)


_P = 16
_TP = 2048
_TF = 2048
_SUB = _TN // _P
_CH = 8


def _dist_argmin_body(fbf_ref, ebf_ref, fsq_ref, c_ref, idx_ref):
    fsq_row = fsq_ref[...]
    acc_v = jnp.full((_CH, _TM), jnp.inf, jnp.float32)
    acc_i = jnp.zeros((_CH, _TM), jnp.int32)
    for p in range(_P):
        s2 = lax.dot_general(ebf_ref[p * _SUB:(p + 1) * _SUB, :], fbf_ref[...],
                             (((1,), (1,)), ((), ())),
                             preferred_element_type=jnp.float32)
        c_p = c_ref[p * _SUB:(p + 1) * _SUB, :]
        for k in range(_SUB // _CH):
            d = fsq_row - s2[k * _CH:(k + 1) * _CH, :] \
                + c_p[k * _CH:(k + 1) * _CH, :]
            row = p * (_SUB // _CH) + k
            lt = d < acc_v
            acc_i = jnp.where(lt, row, acc_i)
            acc_v = jnp.minimum(d, acc_v)
    sub_iota = lax.broadcasted_iota(jnp.int32, (_CH, _TM), 0)
    fidx = acc_i * _CH + sub_iota
    mv = jnp.min(acc_v, axis=0, keepdims=True)
    cand = jnp.where(acc_v == mv, fidx, _NE)
    idx_ref[...] = jnp.min(cand, axis=0, keepdims=True)


def _finalize_body(q_ref, f_ref, out_ref, dsum_ref, acc_ref):
    i = pl.program_id(0)
    q = q_ref[...]
    f = f_ref[...]
    t = q - f
    q1 = f + t
    out_ref[...] = (q + q1) / 2.0

    @pl.when(i == 0)
    def _():
        acc_ref[0, 0] = 0.0

    acc_ref[0, 0] += jnp.sum(t * t)

    @pl.when(i == pl.num_programs(0) - 1)
    def _():
        v = acc_ref[0, 0] / float(_NT * _D)
        dsum_ref[...] = jnp.full((1, 1), v, jnp.float32)


def _build(interpret=False):
    prep = pl.pallas_call(
        _prep_body,
        grid=(_NT // _TP,),
        in_specs=[
            pl.BlockSpec((_TP, _D), lambda i: (i, 0)),
            pl.BlockSpec((_TP, _D), lambda i: (i, 0)),
        ],
        out_specs=[
            pl.BlockSpec((_TP, _D), lambda i: (i, 0)),
            pl.BlockSpec((1, _TP), lambda i: (0, i)),
            pl.BlockSpec((_TP, _D), lambda i: (i, 0)),
            pl.BlockSpec((_TP, 1), lambda i: (i, 0)),
        ],
        out_shape=[
            jax.ShapeDtypeStruct((_NT, _D), jnp.bfloat16),
            jax.ShapeDtypeStruct((1, _NT), jnp.float32),
            jax.ShapeDtypeStruct((_NE, _D), jnp.bfloat16),
            jax.ShapeDtypeStruct((_NE, 1), jnp.float32),
        ],
        interpret=interpret,
    )
    dist_argmin = pl.pallas_call(
        _dist_argmin_body,
        grid=(_NI,),
        in_specs=[
            pl.BlockSpec((_TM, _D), lambda i: (i, 0)),
            pl.BlockSpec((_TN, _D), lambda i: (0, 0)),
            pl.BlockSpec((1, _TM), lambda i: (0, i)),
            pl.BlockSpec((_TN, 1), lambda i: (0, 0)),
        ],
        out_specs=pl.BlockSpec((1, _TM), lambda i: (0, i)),
        out_shape=jax.ShapeDtypeStruct((1, _NT), jnp.int32),
        compiler_params=pltpu.CompilerParams(
            dimension_semantics=("arbitrary",)),
        interpret=interpret,
    )
    finalize = pl.pallas_call(
        _finalize_body,
        grid=(_NT // _TF,),
        in_specs=[
            pl.BlockSpec((_TF, _D), lambda i: (i, 0)),
            pl.BlockSpec((_TF, _D), lambda i: (i, 0)),
        ],
        out_specs=[
            pl.BlockSpec((_TF, _D), lambda i: (i, 0)),
            pl.BlockSpec((1, 1), lambda i: (0, 0)),
        ],
        out_shape=[
            jax.ShapeDtypeStruct((_NT, _D), jnp.float32),
            jax.ShapeDtypeStruct((1, 1), jnp.float32),
        ],
        scratch_shapes=[pltpu.SMEM((1, 1), jnp.float32)],
        interpret=interpret,
    )
    return prep, dist_argmin, finalize


_prep, _dist_argmin, _finalize = _build()


@functools.lru_cache(maxsize=None)
def _make_sc_gather():
    info = plsc.get_sparse_core_info()
    nc, ns = info.num_cores, info.num_subcores
    nw = nc * ns
    bpw = _NT // nw
    mesh = plsc.VectorSubcoreMesh(core_axis_name="c", subcore_axis_name="s")

    nck = 4
    ck = bpw // nck

    @functools.partial(
        pl.kernel, mesh=mesh,
        out_type=jax.ShapeDtypeStruct((_NT, _D), jnp.float32),
        scratch_types=[
            pltpu.VMEM((bpw,), jnp.int32),
            pltpu.VMEM((bpw, _D), jnp.float32),
            pltpu.SemaphoreType.DMA,
            pltpu.SemaphoreType.DMA,
            pltpu.SemaphoreType.DMA,
        ],
    )
    def gather(table_hbm, idx_hbm, out_hbm, idx_v, rows_v, s0, s1, sw):
        wid = lax.axis_index("s") * nc + lax.axis_index("c")
        base = wid * bpw
        pltpu.sync_copy(idx_hbm.at[pl.ds(base, bpw)], idx_v)
        sems = [s0, s1]
        gcp = []
        for t in range(nck):
            gcp.append(pltpu.make_async_copy(
                table_hbm.at[idx_v.at[pl.ds(t * ck, ck)]],
                rows_v.at[pl.ds(t * ck, ck)], sems[t % 2]))
        wcp = [pltpu.make_async_copy(
            rows_v.at[pl.ds(t * ck, ck)],
            out_hbm.at[pl.ds(base + t * ck, ck)], sw) for t in range(nck)]
        gcp[0].start()
        gcp[1].start()
        for t in range(nck):
            gcp[t].wait()
            wcp[t].start()
            if t + 2 < nck:
                gcp[t + 2].start()
        for t in range(nck):
            wcp[t].wait()

    return gather


def kernel(input, embed_weight):
    flatten = input.reshape(_NT, _D)
    fbf, fsq, ebf, c = _prep(flatten, embed_weight)
    idx = _dist_argmin(fbf, ebf, fsq, c)
    quant = _make_sc_gather()(embed_weight, idx.reshape(_NT))
    out, dsum = _finalize(quant, flatten)
    return (out.reshape(input.shape), dsum[0, 0])


# speedup vs baseline: 1.0430x; 1.0430x over previous
import functools

import jax
import jax.numpy as jnp
from jax import lax
from jax.experimental import pallas as pl
from jax.experimental.pallas import tpu as pltpu
from jax.experimental.pallas import tpu_sc as plsc

_D = 256
_NE = 8192
_NT = 8192
_TM = 1024
_TN = 8192
_NI = _NT // _TM
_NJ = _NE // _TN


def _prep_body(f_ref, e_ref, fbf_ref, fsq_ref, ebf_ref, c_ref):
    f = f_ref[...]
    e = e_ref[...]
    norm = jnp.sqrt(jnp.sum(e * e, axis=1, keepdims=True))
    ehat = e / norm
    ebf_ref[...] = (ehat + ehat).astype(jnp.bfloat16)
    c_ref[...] = jnp.sum(ehat * ehat, axis=1, keepdims=True)
    fbf_ref[...] = f.astype(jnp.bfloat16)
    ones = jnp.ones((1, _D), dtype=jnp.float32)
    fsq_ref[...] = lax.dot_general(ones, f * f, (((1,), (1,)), ((), ())),
                                   preferred_element_type=jnp.float32,
                                   precision=lax.Precision.HIGHEST)


_P = 16
_TP = 2048
_TF = 2048
_SUB = _TN // _P
_CH = 8


def _dist_argmin_body(f_ref, e_ref, idx_ref, ebf_s, c_s):
    i = pl.program_id(0)

    @pl.when(i == 0)
    def _():
        e = e_ref[...]
        norm = jnp.sqrt(jnp.sum(e * e, axis=1, keepdims=True))
        ehat = e / norm
        ebf_s[...] = (ehat + ehat).astype(jnp.bfloat16)
        c_s[...] = jnp.sum(ehat * ehat, axis=1, keepdims=True)

    f = f_ref[...]
    fbf = f.astype(jnp.bfloat16)
    ones = jnp.ones((1, _D), dtype=jnp.float32)
    fsq_row = lax.dot_general(ones, f * f, (((1,), (1,)), ((), ())),
                              preferred_element_type=jnp.float32,
                              precision=lax.Precision.HIGHEST)
    acc_v = jnp.full((_CH, _TM), jnp.inf, jnp.float32)
    acc_i = jnp.zeros((_CH, _TM), jnp.int32)
    for p in range(_P):
        s2 = lax.dot_general(ebf_s[p * _SUB:(p + 1) * _SUB, :], fbf,
                             (((1,), (1,)), ((), ())),
                             preferred_element_type=jnp.float32)
        c_p = c_s[p * _SUB:(p + 1) * _SUB, :]
        for k in range(_SUB // _CH):
            d = fsq_row - s2[k * _CH:(k + 1) * _CH, :] \
                + c_p[k * _CH:(k + 1) * _CH, :]
            row = p * (_SUB // _CH) + k
            lt = d < acc_v
            acc_i = jnp.where(lt, row, acc_i)
            acc_v = jnp.minimum(d, acc_v)
    sub_iota = lax.broadcasted_iota(jnp.int32, (_CH, _TM), 0)
    fidx = acc_i * _CH + sub_iota
    mv = jnp.min(acc_v, axis=0, keepdims=True)
    cand = jnp.where(acc_v == mv, fidx, _NE)
    idx_ref[...] = jnp.min(cand, axis=0, keepdims=True)


def _finalize_body(q_ref, f_ref, out_ref, dsum_ref, acc_ref):
    i = pl.program_id(0)
    q = q_ref[...]
    f = f_ref[...]
    t = q - f
    q1 = f + t
    out_ref[...] = (q + q1) / 2.0

    @pl.when(i == 0)
    def _():
        acc_ref[0, 0] = 0.0

    acc_ref[0, 0] += jnp.sum(t * t)

    @pl.when(i == pl.num_programs(0) - 1)
    def _():
        v = acc_ref[0, 0] / float(_NT * _D)
        dsum_ref[...] = jnp.full((1, 1), v, jnp.float32)


def _build(interpret=False):
    dist_argmin = pl.pallas_call(
        _dist_argmin_body,
        grid=(_NI,),
        in_specs=[
            pl.BlockSpec((_TM, _D), lambda i: (i, 0)),
            pl.BlockSpec((_NE, _D), lambda i: (0, 0)),
        ],
        out_specs=pl.BlockSpec((1, _TM), lambda i: (0, i)),
        out_shape=jax.ShapeDtypeStruct((1, _NT), jnp.int32),
        scratch_shapes=[
            pltpu.VMEM((_NE, _D), jnp.bfloat16),
            pltpu.VMEM((_NE, 1), jnp.float32),
        ],
        compiler_params=pltpu.CompilerParams(
            dimension_semantics=("arbitrary",)),
        interpret=interpret,
    )
    finalize = pl.pallas_call(
        _finalize_body,
        grid=(_NT // _TF,),
        in_specs=[
            pl.BlockSpec((_TF, _D), lambda i: (i, 0)),
            pl.BlockSpec((_TF, _D), lambda i: (i, 0)),
        ],
        out_specs=[
            pl.BlockSpec((_TF, _D), lambda i: (i, 0)),
            pl.BlockSpec((1, 1), lambda i: (0, 0)),
        ],
        out_shape=[
            jax.ShapeDtypeStruct((_NT, _D), jnp.float32),
            jax.ShapeDtypeStruct((1, 1), jnp.float32),
        ],
        scratch_shapes=[pltpu.SMEM((1, 1), jnp.float32)],
        interpret=interpret,
    )
    return dist_argmin, finalize


_dist_argmin, _finalize = _build()


@functools.lru_cache(maxsize=None)
def _make_sc_gather():
    info = plsc.get_sparse_core_info()
    nc, ns = info.num_cores, info.num_subcores
    nw = nc * ns
    bpw = _NT // nw
    mesh = plsc.VectorSubcoreMesh(core_axis_name="c", subcore_axis_name="s")

    @functools.partial(
        pl.kernel, mesh=mesh,
        out_type=jax.ShapeDtypeStruct((_NT, _D), jnp.float32),
        scratch_types=[
            pltpu.VMEM((bpw,), jnp.int32),
            pltpu.VMEM((bpw, _D), jnp.float32),
            pltpu.SemaphoreType.DMA,
        ],
    )
    def gather(table_hbm, idx_hbm, out_hbm, idx_v, rows_v, sem):
        wid = lax.axis_index("s") * nc + lax.axis_index("c")
        base = wid * bpw
        pltpu.sync_copy(idx_hbm.at[pl.ds(base, bpw)], idx_v)
        pltpu.async_copy(table_hbm.at[idx_v], rows_v, sem).wait()
        pltpu.sync_copy(rows_v, out_hbm.at[pl.ds(base, bpw)])

    return gather


def kernel(input, embed_weight):
    flatten = input.reshape(_NT, _D)
    idx = _dist_argmin(flatten, embed_weight)
    quant = _make_sc_gather()(embed_weight, idx.reshape(_NT))
    out, dsum = _finalize(quant, flatten)
    return (out.reshape(input.shape), dsum[0, 0])


# speedup vs baseline: 1.0529x; 1.0095x over previous
import functools

import jax
import jax.numpy as jnp
from jax import lax
from jax.experimental import pallas as pl
from jax.experimental.pallas import tpu as pltpu
from jax.experimental.pallas import tpu_sc as plsc

_D = 256
_NE = 8192
_NT = 8192
_TM = 1024
_TN = 8192
_NI = _NT // _TM
_NJ = _NE // _TN


def _prep_body(f_ref, e_ref, fbf_ref, fsq_ref, ebf_ref, c_ref):
    f = f_ref[...]
    e = e_ref[...]
    norm = jnp.sqrt(jnp.sum(e * e, axis=1, keepdims=True))
    ehat = e / norm
    ebf_ref[...] = (ehat + ehat).astype(jnp.bfloat16)
    c_ref[...] = jnp.sum(ehat * ehat, axis=1, keepdims=True)
    fbf_ref[...] = f.astype(jnp.bfloat16)
    ones = jnp.ones((1, _D), dtype=jnp.float32)
    fsq_ref[...] = lax.dot_general(ones, f * f, (((1,), (1,)), ((), ())),
                                   preferred_element_type=jnp.float32,
                                   precision=lax.Precision.HIGHEST)


_P = 16
_TP = 2048
_TF = 4096
_SUB = _TN // _P
_CH = 8


def _dist_argmin_body(f_ref, e_ref, idx_ref, ebf_s, c_s):
    i = pl.program_id(0)

    @pl.when(i == 0)
    def _():
        e = e_ref[...]
        norm = jnp.sqrt(jnp.sum(e * e, axis=1, keepdims=True))
        ehat = e / norm
        ebf_s[...] = (ehat + ehat).astype(jnp.bfloat16)
        c_s[...] = jnp.sum(ehat * ehat, axis=1, keepdims=True)

    f = f_ref[...]
    fbf = f.astype(jnp.bfloat16)
    ones = jnp.ones((1, _D), dtype=jnp.float32)
    fsq_row = lax.dot_general(ones, f * f, (((1,), (1,)), ((), ())),
                              preferred_element_type=jnp.float32,
                              precision=lax.Precision.HIGHEST)
    acc_v = jnp.full((_CH, _TM), jnp.inf, jnp.float32)
    acc_i = jnp.zeros((_CH, _TM), jnp.int32)
    for p in range(_P):
        s2 = lax.dot_general(ebf_s[p * _SUB:(p + 1) * _SUB, :], fbf,
                             (((1,), (1,)), ((), ())),
                             preferred_element_type=jnp.float32)
        c_p = c_s[p * _SUB:(p + 1) * _SUB, :]
        for k in range(_SUB // _CH):
            d = fsq_row - s2[k * _CH:(k + 1) * _CH, :] \
                + c_p[k * _CH:(k + 1) * _CH, :]
            row = p * (_SUB // _CH) + k
            lt = d < acc_v
            acc_i = jnp.where(lt, row, acc_i)
            acc_v = jnp.minimum(d, acc_v)
    sub_iota = lax.broadcasted_iota(jnp.int32, (_CH, _TM), 0)
    fidx = acc_i * _CH + sub_iota
    mv = jnp.min(acc_v, axis=0, keepdims=True)
    cand = jnp.where(acc_v == mv, fidx, _NE)
    idx_ref[...] = jnp.min(cand, axis=0, keepdims=True)


def _finalize_body(q_ref, f_ref, out_ref, dsum_ref, acc_ref):
    i = pl.program_id(0)
    q = q_ref[...]
    f = f_ref[...]
    t = q - f
    q1 = f + t
    out_ref[...] = (q + q1) / 2.0

    @pl.when(i == 0)
    def _():
        acc_ref[0, 0] = 0.0

    acc_ref[0, 0] += jnp.sum(t * t)

    @pl.when(i == pl.num_programs(0) - 1)
    def _():
        v = acc_ref[0, 0] / float(_NT * _D)
        dsum_ref[...] = jnp.full((1, 1), v, jnp.float32)


def _build(interpret=False):
    dist_argmin = pl.pallas_call(
        _dist_argmin_body,
        grid=(_NI,),
        in_specs=[
            pl.BlockSpec((_TM, _D), lambda i: (i, 0)),
            pl.BlockSpec((_NE, _D), lambda i: (0, 0)),
        ],
        out_specs=pl.BlockSpec((1, _TM), lambda i: (0, i)),
        out_shape=jax.ShapeDtypeStruct((1, _NT), jnp.int32),
        scratch_shapes=[
            pltpu.VMEM((_NE, _D), jnp.bfloat16),
            pltpu.VMEM((_NE, 1), jnp.float32),
        ],
        compiler_params=pltpu.CompilerParams(
            dimension_semantics=("arbitrary",)),
        interpret=interpret,
    )
    finalize = pl.pallas_call(
        _finalize_body,
        grid=(_NT // _TF,),
        in_specs=[
            pl.BlockSpec((_TF, _D), lambda i: (i, 0)),
            pl.BlockSpec((_TF, _D), lambda i: (i, 0)),
        ],
        out_specs=[
            pl.BlockSpec((_TF, _D), lambda i: (i, 0)),
            pl.BlockSpec((1, 1), lambda i: (0, 0)),
        ],
        out_shape=[
            jax.ShapeDtypeStruct((_NT, _D), jnp.float32),
            jax.ShapeDtypeStruct((1, 1), jnp.float32),
        ],
        scratch_shapes=[pltpu.SMEM((1, 1), jnp.float32)],
        interpret=interpret,
    )
    return dist_argmin, finalize


_dist_argmin, _finalize = _build()


@functools.lru_cache(maxsize=None)
def _make_sc_gather():
    info = plsc.get_sparse_core_info()
    nc, ns = info.num_cores, info.num_subcores
    nw = nc * ns
    bpw = _NT // nw
    mesh = plsc.VectorSubcoreMesh(core_axis_name="c", subcore_axis_name="s")

    @functools.partial(
        pl.kernel, mesh=mesh,
        out_type=jax.ShapeDtypeStruct((_NT, _D), jnp.float32),
        scratch_types=[
            pltpu.VMEM((bpw,), jnp.int32),
            pltpu.VMEM((bpw, _D), jnp.float32),
            pltpu.SemaphoreType.DMA,
        ],
    )
    def gather(table_hbm, idx_hbm, out_hbm, idx_v, rows_v, sem):
        wid = lax.axis_index("s") * nc + lax.axis_index("c")
        base = wid * bpw
        pltpu.sync_copy(idx_hbm.at[pl.ds(base, bpw)], idx_v)
        pltpu.async_copy(table_hbm.at[idx_v], rows_v, sem).wait()
        pltpu.sync_copy(rows_v, out_hbm.at[pl.ds(base, bpw)])

    return gather


def kernel(input, embed_weight):
    flatten = input.reshape(_NT, _D)
    idx = _dist_argmin(flatten, embed_weight)
    quant = _make_sc_gather()(embed_weight, idx.reshape(_NT))
    out, dsum = _finalize(quant, flatten)
    return (out.reshape(input.shape), dsum[0, 0])
